# baseline (speedup 1.0000x reference)
import functools

import jax
import jax.numpy as jnp
from jax import lax
from jax.experimental import pallas as pl
from jax.experimental.pallas import tpu as pltpu
from jax.experimental.pallas import tpu_sc as plsc

N = 100000
D = 128
ROWS_PER_DEG = 16000
NUM_WORKERS = 32
NQ = 4
CHUNK = 125
PADC = 128
NUM_COLS = 21
NBUF = 6
_OFF = (0, 1, 3, 6, 10, 15)


def _acc_add(acc, buf, first):
    def body(r, carry):
        for cc in range(D // 16):
            sl = pl.ds(cc * 16, 16)
            if first:
                acc[r, sl] = buf[r, sl]
            else:
                plsc.addupdate(acc.at[r, sl], buf[r, sl])
        return carry
    lax.fori_loop(0, PADC, body, 0, unroll=2)


def _sc_gather_sum(nf, idx):
    mesh = plsc.VectorSubcoreMesh(core_axis_name="c", subcore_axis_name="s")

    @functools.partial(
        pl.kernel,
        out_type=jax.ShapeDtypeStruct((NQ * 6 * NUM_WORKERS, CHUNK, D),
                                      jnp.float32),
        mesh=mesh,
        scratch_types=[
            pltpu.VMEM((NQ * NUM_COLS, PADC), jnp.int32),
            pltpu.VMEM((PADC, D), jnp.float32),
        ] + [pltpu.VMEM((PADC, D), jnp.float32) for _ in range(NBUF)]
          + [pltpu.SemaphoreType.DMA for _ in range(NBUF + 1)],
    )
    def k(nf_hbm, idx_hbm, out_hbm, idx_v, acc, *rest):
        bufs = rest[:NBUF]
        sems = rest[NBUF:2 * NBUF]
        sem_st = rest[2 * NBUF]
        cid = lax.axis_index("c")
        sid = lax.axis_index("s")
        wid = sid * 2 + cid
        pltpu.sync_copy(idx_hbm.at[wid], idx_v)

        gathers = []
        for d in range(1, 7):
            c0 = _OFF[d - 1]
            for q in range(NQ):
                for j in range(d):
                    gathers.append((NQ * (c0 + j) + q, j == 0,
                                    (d, q) if j == d - 1 else None))
        ngather = len(gathers)

        cps = [None] * ngather
        state = {"nfire": 0, "store": None}

        def fire(i):
            row, _, _ = gathers[i]
            cps[i] = pltpu.async_copy(
                nf_hbm.at[idx_v.at[row]], bufs[i % NBUF], sems[i % NBUF])

        for i, (row, is_first, task_end) in enumerate(gathers):
            while state["nfire"] < ngather and state["nfire"] <= i + NBUF - 1:
                fire(state["nfire"])
                state["nfire"] += 1
            cps[i].wait()
            if is_first and state["store"] is not None:
                state["store"].wait()
            _acc_add(acc, bufs[i % NBUF], first=is_first)
            if task_end is not None:
                d, q = task_end
                blk = ((d - 1) * NUM_WORKERS + wid) * NQ + q
                state["store"] = pltpu.async_copy(
                    acc.at[pl.ds(0, CHUNK)], out_hbm.at[blk], sem_st)
        state["store"].wait()

    return k(nf, idx)


def _tc_linear(nf, nsum, W, b):
    BS = 4000
    nblocks = N // BS

    def ws_idx(g):
        return (jnp.where(g == 0, 0, 2 * ((g + 3) // 4)), 0, 0)

    def wn_idx(g):
        return (jnp.maximum(2 * ((g + 3) // 4) - 1, 0), 0, 0)

    def body(x_ref, ns_ref, ws_ref, wn_ref, bs_ref, bn_ref, o_ref):
        g = pl.program_id(0)
        o_ref[...] = jnp.dot(
            x_ref[...], ws_ref[0], preferred_element_type=jnp.float32,
            precision=lax.Precision.HIGHEST) + bs_ref[0, 0]

        @pl.when(g > 0)
        def _():
            o_ref[...] += jnp.dot(
                ns_ref[...], wn_ref[0], preferred_element_type=jnp.float32,
                precision=lax.Precision.HIGHEST) + bn_ref[0, 0]

    br = b.reshape(b.shape[0], 1, D)
    return pl.pallas_call(
        body,
        grid=(nblocks,),
        in_specs=[
            pl.BlockSpec((BS, D), lambda g: (g, 0)),
            pl.BlockSpec((BS, D), lambda g: (jnp.maximum(g - 1, 0), 0)),
            pl.BlockSpec((1, D, D), ws_idx),
            pl.BlockSpec((1, D, D), wn_idx),
            pl.BlockSpec((1, 1, D), ws_idx),
            pl.BlockSpec((1, 1, D), wn_idx),
        ],
        out_specs=pl.BlockSpec((BS, D), lambda g: (g, 0)),
        out_shape=jax.ShapeDtypeStruct((N, D), jnp.float32),
    )(nf, nsum, W, W, br, br)


def kernel(node_features, deg_slice, deg_adj_1, deg_adj_2, deg_adj_3,
           deg_adj_4, deg_adj_5, deg_adj_6, W, b):
    adjs = (deg_adj_1, deg_adj_2, deg_adj_3, deg_adj_4, deg_adj_5, deg_adj_6)
    cols = jnp.concatenate([a.astype(jnp.int32).T for a in adjs], axis=0)
    idx = cols.reshape(NUM_COLS, NUM_WORKERS, NQ, CHUNK)
    idx = jnp.pad(idx, ((0, 0), (0, 0), (0, 0), (0, PADC - CHUNK)))
    idx = idx.transpose(1, 0, 2, 3).reshape(NUM_WORKERS, NQ * NUM_COLS, PADC)
    nsum = _sc_gather_sum(node_features, idx)
    nsum = nsum.reshape(6 * ROWS_PER_DEG, D)
    return _tc_linear(node_features, nsum, W, b)


# speedup vs baseline: 1.5469x; 1.5469x over previous
import functools

import jax
import jax.numpy as jnp
from jax import lax
from jax.experimental import pallas as pl
from jax.experimental.pallas import tpu as pltpu
from jax.experimental.pallas import tpu_sc as plsc

N = 100000
D = 128
ROWS_PER_DEG = 16000
NUM_WORKERS = 32
IW = 128
_PARAMS = {1: (128, 4), 2: (64, 8), 3: (40, 14), 4: (32, 16),
           5: (24, 22), 6: (16, 32)}
_ROWBASE = {1: 0, 2: 4, 3: 12, 4: 26, 5: 42, 6: 64}
_NSTREAM = 96


def _reduce(d, R, g, ob):
    def row_body(r, carry):
        base = r * d
        for cc in range(D // 16):
            sl = pl.ds(cc * 16, 16)
            v = g[base, sl]
            for j in range(1, d):
                v = v + g[base + j, sl]
            ob[r, sl] = v
        return carry
    lax.fori_loop(0, R, row_body, 0, unroll=2 if d < 4 else 1)


def _sc_gather_sum(nf, idx):
    mesh = plsc.VectorSubcoreMesh(core_axis_name="c", subcore_axis_name="s")

    @functools.partial(
        pl.kernel,
        out_type=jax.ShapeDtypeStruct((6 * ROWS_PER_DEG, D), jnp.float32),
        mesh=mesh,
        scratch_types=[
            pltpu.VMEM((_NSTREAM, IW), jnp.int32),
            pltpu.VMEM((IW, D), jnp.float32),
            pltpu.VMEM((IW, D), jnp.float32),
            pltpu.VMEM((IW, D), jnp.float32),
            pltpu.VMEM((IW, D), jnp.float32),
            pltpu.SemaphoreType.DMA,
            pltpu.SemaphoreType.DMA,
            pltpu.SemaphoreType.DMA,
            pltpu.SemaphoreType.DMA,
        ],
    )
    def k(nf_hbm, idx_hbm, out_hbm, idx_v, g0, g1, o0, o1,
          gs0, gs1, os0, os1):
        cid = lax.axis_index("c")
        sid = lax.axis_index("s")
        wid = sid * 2 + cid
        pltpu.sync_copy(idx_hbm.at[wid], idx_v)
        gbufs, gsems = (g0, g1), (gs0, gs1)
        obufs, osems = (o0, o1), (os0, os1)

        def store_row0(d, s):
            R, S = _PARAMS[d]
            return ((d - 1) * ROWS_PER_DEG
                    + jnp.minimum((wid * S + s) * R, ROWS_PER_DEG - R))

        R1, S1 = _PARAMS[1]
        bufs4 = (g0, g1, o0, o1)
        sems4 = (gs0, gs1, os0, os1)
        cps = [pltpu.async_copy(nf_hbm.at[idx_v.at[s]], bufs4[s], sems4[s])
               for s in range(S1)]
        sts = []
        for s in range(S1):
            cps[s].wait()
            sts.append(pltpu.async_copy(
                bufs4[s], out_hbm.at[pl.ds(store_row0(1, s), R1)], sems4[s]))
        for s in range(S1):
            sts[s].wait()

        for d in range(2, 7):
            R, S = _PARAMS[d]
            L = R * d
            rowbase = _ROWBASE[d]

            def fire_gather(s, b, L=L, rowbase=rowbase):
                return pltpu.async_copy(
                    nf_hbm.at[idx_v.at[rowbase + s, pl.ds(0, L)]],
                    gbufs[b].at[pl.ds(0, L)], gsems[b])

            def task(s, b, first, last, d=d, R=R, L=L):
                pltpu.make_async_copy(
                    nf_hbm.at[pl.ds(0, L)], gbufs[b].at[pl.ds(0, L)],
                    gsems[b]).wait()
                if not first:
                    pltpu.make_async_copy(
                        obufs[b].at[pl.ds(0, R)],
                        nf_hbm.at[pl.ds(0, R)], osems[b]).wait()
                _reduce(d, R, gbufs[b], obufs[b])
                pltpu.async_copy(
                    obufs[b].at[pl.ds(0, R)],
                    out_hbm.at[pl.ds(store_row0(d, s), R)], osems[b])
                if not last:
                    fire_gather(s + 2, b)

            fire_gather(0, 0)
            fire_gather(1, 1)
            task(0, 0, first=True, last=False)
            task(1, 1, first=True, last=False)

            def outer(o, carry, task=task):
                s0 = 2 * o
                task(s0, 0, first=False, last=False)
                task(s0 + 1, 1, first=False, last=False)
                return carry
            lax.fori_loop(1, S // 2 - 1, outer, 0)

            task(S - 2, 0, first=False, last=True)
            task(S - 1, 1, first=False, last=True)
            for b in range(2):
                pltpu.make_async_copy(
                    obufs[b].at[pl.ds(0, R)],
                    nf_hbm.at[pl.ds(0, R)], osems[b]).wait()

    return k(nf, idx)


def _tc_linear(nf, nsum, W, b):
    BS = 4000
    nblocks = N // BS

    def ws_idx(g):
        return (jnp.where(g == 0, 0, 2 * ((g + 3) // 4)), 0, 0)

    def wn_idx(g):
        return (jnp.maximum(2 * ((g + 3) // 4) - 1, 0), 0, 0)

    def body(x_ref, ns_ref, ws_ref, wn_ref, bs_ref, bn_ref, o_ref):
        g = pl.program_id(0)
        o_ref[...] = jnp.dot(
            x_ref[...], ws_ref[0], preferred_element_type=jnp.float32,
            precision=lax.Precision.HIGHEST) + bs_ref[0, 0]

        @pl.when(g > 0)
        def _():
            o_ref[...] += jnp.dot(
                ns_ref[...], wn_ref[0], preferred_element_type=jnp.float32,
                precision=lax.Precision.HIGHEST) + bn_ref[0, 0]

    br = b.reshape(b.shape[0], 1, D)
    return pl.pallas_call(
        body,
        grid=(nblocks,),
        in_specs=[
            pl.BlockSpec((BS, D), lambda g: (g, 0)),
            pl.BlockSpec((BS, D), lambda g: (jnp.maximum(g - 1, 0), 0)),
            pl.BlockSpec((1, D, D), ws_idx),
            pl.BlockSpec((1, D, D), wn_idx),
            pl.BlockSpec((1, 1, D), ws_idx),
            pl.BlockSpec((1, 1, D), wn_idx),
        ],
        out_specs=pl.BlockSpec((BS, D), lambda g: (g, 0)),
        out_shape=jax.ShapeDtypeStruct((N, D), jnp.float32),
    )(nf, nsum, W, W, br, br)


def _build_idx(adjs):
    per_deg = []
    for d in range(1, 7):
        R, S = _PARAMS[d]
        G = NUM_WORKERS * S
        flat = adjs[d - 1].astype(jnp.int32).reshape(ROWS_PER_DEG * d)
        tfull = ROWS_PER_DEG // R
        parts = [flat[:tfull * R * d].reshape(tfull, R * d)]
        if G > tfull:
            tail = flat[(ROWS_PER_DEG - R) * d:].reshape(1, R * d)
            parts.append(jnp.tile(tail, (G - tfull, 1)))
        rows = jnp.concatenate(parts, axis=0)
        rows = jnp.pad(rows, ((0, 0), (0, IW - R * d)))
        per_deg.append(rows.reshape(NUM_WORKERS, S, IW))
    return jnp.concatenate(per_deg, axis=1)


def kernel(node_features, deg_slice, deg_adj_1, deg_adj_2, deg_adj_3,
           deg_adj_4, deg_adj_5, deg_adj_6, W, b):
    adjs = (deg_adj_1, deg_adj_2, deg_adj_3, deg_adj_4, deg_adj_5, deg_adj_6)
    idx = _build_idx(adjs)
    nsum = _sc_gather_sum(node_features, idx)
    return _tc_linear(node_features, nsum, W, b)


# speedup vs baseline: 1.9677x; 1.2720x over previous
import functools

import jax
import jax.numpy as jnp
from jax import lax
from jax.experimental import pallas as pl
from jax.experimental.pallas import tpu as pltpu
from jax.experimental.pallas import tpu_sc as plsc

N = 100000
D = 128
ROWS_PER_DEG = 16000
NUM_WORKERS = 32
IW = 128
_PARAMS = {1: (128, 4), 2: (64, 8), 3: (40, 14), 4: (32, 16),
           5: (24, 22), 6: (16, 32)}
_SECBASE = {1: 0, 2: 512, 3: 1536, 4: 3216, 5: 5264, 6: 7904}
_IDXWORDS = 10976


def _reduce(d, R, g, ob):
    def row_body(r, carry):
        base = r * d
        for cc in range(D // 16):
            sl = pl.ds(cc * 16, 16)
            v = g[base, sl]
            for j in range(1, d):
                v = v + g[base + j, sl]
            ob[r, sl] = v
        return carry
    lax.fori_loop(0, R, row_body, 0, unroll=2 if d < 4 else 1)


def _sc_gather_sum(nf, adj_flats):
    mesh = plsc.VectorSubcoreMesh(core_axis_name="c", subcore_axis_name="s")

    @functools.partial(
        pl.kernel,
        out_type=jax.ShapeDtypeStruct((6 * ROWS_PER_DEG, D), jnp.float32),
        mesh=mesh,
        scratch_types=[
            pltpu.VMEM((_IDXWORDS,), jnp.int32),
            pltpu.VMEM((IW, D), jnp.float32),
            pltpu.VMEM((IW, D), jnp.float32),
            pltpu.VMEM((IW, D), jnp.float32),
            pltpu.VMEM((IW, D), jnp.float32),
            pltpu.SemaphoreType.DMA,
            pltpu.SemaphoreType.DMA,
            pltpu.SemaphoreType.DMA,
            pltpu.SemaphoreType.DMA,
            pltpu.SemaphoreType.DMA,
        ],
    )
    def k(nf_hbm, a1, a2, a3, a4, a5, a6, out_hbm, idx_v, g0, g1, o0, o1,
          sem_idx, gs0, gs1, os0, os1):
        adj_refs = (a1, a2, a3, a4, a5, a6)
        cid = lax.axis_index("c")
        sid = lax.axis_index("s")
        wid = sid * 2 + cid
        gbufs, gsems = (g0, g1), (gs0, gs1)
        obufs, osems = (o0, o1), (os0, os1)

        w0s = {}
        slab_cps = []
        for d in range(1, 7):
            R, S = _PARAMS[d]
            w0s[d] = jnp.minimum(wid * S * R, ROWS_PER_DEG - S * R)
            n = S * R * d
            slab_cps.append(pltpu.async_copy(
                adj_refs[d - 1].at[pl.ds(w0s[d] * d, n)],
                idx_v.at[pl.ds(_SECBASE[d], n)], sem_idx))
        for cp in slab_cps:
            cp.wait()

        def base_row(d, s):
            R, S = _PARAMS[d]
            return jnp.minimum((wid * S + s) * R, ROWS_PER_DEG - R)

        def idx_slice(d, s):
            R, S = _PARAMS[d]
            off = _SECBASE[d] + (base_row(d, s) - w0s[d]) * d
            return idx_v.at[pl.ds(off, R * d)]

        def store_row0(d, s):
            return (d - 1) * ROWS_PER_DEG + base_row(d, s)

        R1, S1 = _PARAMS[1]
        bufs4 = (g0, g1, o0, o1)
        sems4 = (gs0, gs1, os0, os1)
        cps = [pltpu.async_copy(nf_hbm.at[idx_slice(1, s)], bufs4[s],
                                sems4[s])
               for s in range(S1)]
        sts = []
        for s in range(S1):
            cps[s].wait()
            sts.append(pltpu.async_copy(
                bufs4[s], out_hbm.at[pl.ds(store_row0(1, s), R1)], sems4[s]))
        for s in range(S1):
            sts[s].wait()

        for d in range(2, 7):
            R, S = _PARAMS[d]
            L = R * d

            def fire_gather(s, b, d=d, L=L):
                return pltpu.async_copy(
                    nf_hbm.at[idx_slice(d, s)],
                    gbufs[b].at[pl.ds(0, L)], gsems[b])

            def task(s, b, first, last, d=d, R=R, L=L):
                pltpu.make_async_copy(
                    nf_hbm.at[pl.ds(0, L)], gbufs[b].at[pl.ds(0, L)],
                    gsems[b]).wait()
                if not first:
                    pltpu.make_async_copy(
                        obufs[b].at[pl.ds(0, R)],
                        nf_hbm.at[pl.ds(0, R)], osems[b]).wait()
                _reduce(d, R, gbufs[b], obufs[b])
                pltpu.async_copy(
                    obufs[b].at[pl.ds(0, R)],
                    out_hbm.at[pl.ds(store_row0(d, s), R)], osems[b])
                if not last:
                    fire_gather(s + 2, b)

            fire_gather(0, 0)
            fire_gather(1, 1)
            task(0, 0, first=True, last=False)
            task(1, 1, first=True, last=False)

            def outer(o, carry, task=task):
                s0 = 2 * o
                task(s0, 0, first=False, last=False)
                task(s0 + 1, 1, first=False, last=False)
                return carry
            lax.fori_loop(1, S // 2 - 1, outer, 0)

            task(S - 2, 0, first=False, last=True)
            task(S - 1, 1, first=False, last=True)
            for b in range(2):
                pltpu.make_async_copy(
                    obufs[b].at[pl.ds(0, R)],
                    nf_hbm.at[pl.ds(0, R)], osems[b]).wait()

    return k(nf, *adj_flats)


def _tc_linear(nf, nsum, W, b):
    BS = 4000
    nblocks = N // BS

    def ws_idx(g):
        return (jnp.where(g == 0, 0, 2 * ((g + 3) // 4)), 0, 0)

    def wn_idx(g):
        return (jnp.maximum(2 * ((g + 3) // 4) - 1, 0), 0, 0)

    def body(x_ref, ns_ref, ws_ref, wn_ref, bs_ref, bn_ref, o_ref):
        g = pl.program_id(0)
        o_ref[...] = jnp.dot(
            x_ref[...], ws_ref[0],
            preferred_element_type=jnp.float32) + bs_ref[0, 0]

        @pl.when(g > 0)
        def _():
            o_ref[...] += jnp.dot(
                ns_ref[...], wn_ref[0],
                preferred_element_type=jnp.float32) + bn_ref[0, 0]

    br = b.reshape(b.shape[0], 1, D)
    return pl.pallas_call(
        body,
        grid=(nblocks,),
        in_specs=[
            pl.BlockSpec((BS, D), lambda g: (g, 0)),
            pl.BlockSpec((BS, D), lambda g: (jnp.maximum(g - 1, 0), 0)),
            pl.BlockSpec((1, D, D), ws_idx),
            pl.BlockSpec((1, D, D), wn_idx),
            pl.BlockSpec((1, 1, D), ws_idx),
            pl.BlockSpec((1, 1, D), wn_idx),
        ],
        out_specs=pl.BlockSpec((BS, D), lambda g: (g, 0)),
        out_shape=jax.ShapeDtypeStruct((N, D), jnp.float32),
    )(nf, nsum, W, W, br, br)


def kernel(node_features, deg_slice, deg_adj_1, deg_adj_2, deg_adj_3,
           deg_adj_4, deg_adj_5, deg_adj_6, W, b):
    adjs = (deg_adj_1, deg_adj_2, deg_adj_3, deg_adj_4, deg_adj_5, deg_adj_6)
    adj_flats = tuple(a.astype(jnp.int32).reshape(-1) for a in adjs)
    nsum = _sc_gather_sum(node_features, adj_flats)
    return _tc_linear(node_features, nsum, W, b)
